# baseline (speedup 1.0000x reference)
import functools

import jax
import jax.numpy as jnp
from jax import lax
from jax.experimental import pallas as pl
from jax.experimental.pallas import tpu as pltpu
from jax.experimental.pallas import tpu_sc as plsc

EMB_DIM = 128
N_POS = 128 * 370
NEGATIVE_SIZE = 5
N_NEG = N_POS * NEGATIVE_SIZE

SC_NEG = 75776
TC_NEG = N_NEG - SC_NEG
GRID = 37
POS_BLOCK = N_POS // GRID
NEG_BLOCK = TC_NEG // GRID

NW = 32
SC_R = SC_NEG // NW
SC_C = 32
SC_NCHUNK = SC_R // SC_C
SC_BASE = TC_NEG

_COEF = (6.93186578e-01, 1.24800880e-01, -5.03562042e-03, 2.85647393e-04,
         -1.41225172e-05, 5.10029689e-07, -1.20037919e-08, 1.61785882e-10,
         -9.40153145e-13)


def _tc_loss_kernel(pu_ref, pv_ref, nu_ref, nv_ref, out_ref):
    step = pl.program_id(0)

    def body(u, v, sign):
        n = u.shape[0]
        prod = (u * v).reshape(n // 128, 128, EMB_DIM)
        prod_t = jnp.swapaxes(prod, 1, 2)
        score = jnp.sum(prod_t, axis=1)
        score = jnp.clip(score, -6.0, 6.0)
        return jnp.sum(jnp.log1p(jnp.exp(sign * score)))

    partial = (body(pu_ref[...], pv_ref[...], -1.0)
               + body(nu_ref[...], nv_ref[...], 1.0))

    @pl.when(step == 0)
    def _init():
        out_ref[0, 0] = partial

    @pl.when(step != 0)
    def _acc():
        out_ref[0, 0] += partial


def _tc_call(emb_pos_u, emb_pos_v, emb_neg_u, emb_neg_v):
    pos_spec = pl.BlockSpec((POS_BLOCK, EMB_DIM), lambda i: (i, 0))
    neg_spec = pl.BlockSpec((NEG_BLOCK, EMB_DIM), lambda i: (i, 0))
    return pl.pallas_call(
        _tc_loss_kernel,
        grid=(GRID,),
        in_specs=[pos_spec, pos_spec, neg_spec, neg_spec],
        out_specs=pl.BlockSpec((1, 1), lambda i: (0, 0),
                               memory_space=pltpu.SMEM),
        out_shape=jax.ShapeDtypeStruct((1, 1), jnp.float32),
    )(emb_pos_u, emb_pos_v, emb_neg_u, emb_neg_v)


def _sc_kernel_body(nu_hbm, nv_hbm, out_hbm,
                    ub0, ub1, vb0, vb1, ui0, ui1, vi0, vi1,
                    accv, us0, us1, vs0, vs1):
    ubufs = (ub0, ub1)
    vbufs = (vb0, vb1)
    uidx = (ui0, ui1)
    vidx = (vi0, vi1)
    cid = lax.axis_index("c")
    sid = lax.axis_index("s")
    wid = sid * 2 + cid
    base = SC_BASE + wid * SC_R

    usems = (us0, us1)
    vsems = (vs0, vs1)

    lane16 = lax.iota(jnp.int32, 16)
    sixteen = jnp.full((16,), 16, jnp.int32)
    one = jnp.full((16,), 1, jnp.int32)

    def start(i, b):
        row0 = base + i * SC_C
        rvec = lane16 + jax.lax.broadcast(row0, (16,))
        for h in range(SC_C // 16):
            uidx[b][pl.ds(h * 16, 16)] = rvec
            vidx[b][pl.ds(h * 16, 16)] = rvec
            rvec = rvec + sixteen
        pltpu.make_async_copy(nu_hbm.at[uidx[b]], ubufs[b],
                              usems[b]).start()
        pltpu.make_async_copy(nv_hbm.at[vidx[b]], vbufs[b],
                              vsems[b]).start()

    def wait(b):
        pltpu.make_async_copy(nu_hbm.at[uidx[b]], ubufs[b],
                              usems[b]).wait()
        pltpu.make_async_copy(nv_hbm.at[vidx[b]], vbufs[b],
                              vsems[b]).wait()

    def compute(b, acc):
        for g in range(SC_C // 16):
            rows = lane16 + jnp.full((16,), g * 16, jnp.int32)
            zero = jnp.zeros((16,), jnp.float32)

            def kbody(t, carry):
                cols = carry[0]
                accs = list(carry[1:])
                for kk in range(16):
                    uvec = plsc.load_gather(ubufs[b], [rows, cols])
                    vvec = plsc.load_gather(vbufs[b], [rows, cols])
                    cols = cols + one
                    accs[kk % 8] = accs[kk % 8] + uvec * vvec
                return (cols, *accs)

            carry = lax.fori_loop(0, EMB_DIM // 16, kbody,
                                  (jnp.zeros((16,), jnp.int32),) + (zero,) * 8)
            accs = carry[1:]
            dots = ((accs[0] + accs[1]) + (accs[2] + accs[3])) + (
                (accs[4] + accs[5]) + (accs[6] + accs[7]))
            s = jnp.clip(dots, -6.0, 6.0)
            y = s * s
            p = jnp.full((16,), _COEF[8], jnp.float32)
            for c in _COEF[7::-1]:
                p = p * y + jnp.full((16,), c, jnp.float32)
            acc = acc + (0.5 * s + p)
        return acc

    start(0, 0)
    start(1, 1)

    def body2(j, acc):
        for b in (0, 1):
            i = j * 2 + b
            wait(b)
            acc = compute(b, acc)

            @pl.when(i + 2 < SC_NCHUNK)
            def _():
                start(i + 2, b)
        return acc

    acc = lax.fori_loop(0, SC_NCHUNK // 2, body2, jnp.zeros((16,),
                                                            jnp.float32))
    accv[...] = acc
    pltpu.sync_copy(accv, out_hbm.at[wid])


def _sc_call(emb_neg_u, emb_neg_v):
    mesh = plsc.VectorSubcoreMesh(core_axis_name="c", subcore_axis_name="s",
                                  num_cores=2, num_subcores=16)
    return pl.kernel(
        _sc_kernel_body,
        out_type=jax.ShapeDtypeStruct((NW, 16), jnp.float32),
        mesh=mesh,
        scratch_types=[
            pltpu.VMEM((SC_C, EMB_DIM), jnp.float32),
            pltpu.VMEM((SC_C, EMB_DIM), jnp.float32),
            pltpu.VMEM((SC_C, EMB_DIM), jnp.float32),
            pltpu.VMEM((SC_C, EMB_DIM), jnp.float32),
            pltpu.VMEM((SC_C,), jnp.int32),
            pltpu.VMEM((SC_C,), jnp.int32),
            pltpu.VMEM((SC_C,), jnp.int32),
            pltpu.VMEM((SC_C,), jnp.int32),
            pltpu.VMEM((16,), jnp.float32),
            pltpu.SemaphoreType.DMA,
            pltpu.SemaphoreType.DMA,
            pltpu.SemaphoreType.DMA,
            pltpu.SemaphoreType.DMA,
        ],
        compiler_params=pltpu.CompilerParams(needs_layout_passes=False),
    )(emb_neg_u, emb_neg_v)


def kernel(emb_pos_u, emb_pos_v, emb_neg_u, emb_neg_v):
    sc_out = _sc_call(emb_neg_u, emb_neg_v)
    tc_tot = _tc_call(emb_pos_u, emb_pos_v, emb_neg_u, emb_neg_v)
    return (tc_tot[0, 0] + jnp.sum(sc_out)) / jnp.float32(N_POS)


# speedup vs baseline: 1.7161x; 1.7161x over previous
import functools

import jax
import jax.numpy as jnp
from jax import lax
from jax.experimental import pallas as pl
from jax.experimental.pallas import tpu as pltpu
from jax.experimental.pallas import tpu_sc as plsc

EMB_DIM = 128
N_POS = 128 * 370
NEGATIVE_SIZE = 5
N_NEG = N_POS * NEGATIVE_SIZE

SC_NEG = 102400
TC_NEG = N_NEG - SC_NEG
GRID = 10
POS_BLOCK = N_POS // GRID
NEG_BLOCK = TC_NEG // GRID

NW = 32
SC_R = SC_NEG // NW
GSZ = 16
SC_NGROUP = SC_R // GSZ
DEPTH = 10
SC_BASE = TC_NEG

_COEF = (6.93186578e-01, 1.24800880e-01, -5.03562042e-03, 2.85647393e-04,
         -1.41225172e-05, 5.10029689e-07, -1.20037919e-08, 1.61785882e-10,
         -9.40153145e-13)


def _tc_loss_kernel(pu_ref, pv_ref, nu_ref, nv_ref, out_ref):
    step = pl.program_id(0)

    def body(u, v, sign):
        n = u.shape[0]
        prod = (u * v).reshape(n // 128, 128, EMB_DIM)
        prod_t = jnp.swapaxes(prod, 1, 2)
        score = jnp.sum(prod_t, axis=1)
        score = jnp.clip(score, -6.0, 6.0)
        return jnp.sum(jnp.log1p(jnp.exp(sign * score)))

    partial = (body(pu_ref[...], pv_ref[...], -1.0)
               + body(nu_ref[...], nv_ref[...], 1.0))

    @pl.when(step == 0)
    def _init():
        out_ref[0, 0] = partial

    @pl.when(step != 0)
    def _acc():
        out_ref[0, 0] += partial


def _tc_call(emb_pos_u, emb_pos_v, emb_neg_u, emb_neg_v):
    pos_spec = pl.BlockSpec((POS_BLOCK, EMB_DIM), lambda i: (i, 0))
    neg_spec = pl.BlockSpec((NEG_BLOCK, EMB_DIM), lambda i: (i, 0))
    return pl.pallas_call(
        _tc_loss_kernel,
        grid=(GRID,),
        in_specs=[pos_spec, pos_spec, neg_spec, neg_spec],
        out_specs=pl.BlockSpec((1, 1), lambda i: (0, 0),
                               memory_space=pltpu.SMEM),
        out_shape=jax.ShapeDtypeStruct((1, 1), jnp.float32),
    )(emb_pos_u, emb_pos_v, emb_neg_u, emb_neg_v)


def _sc_kernel_body(nu_hbm, nv_hbm, out_hbm, *scratch):
    ubufs = scratch[:DEPTH]
    vbufs = scratch[DEPTH:2 * DEPTH]
    accv = scratch[2 * DEPTH]
    sems = scratch[2 * DEPTH + 1:]
    cid = lax.axis_index("c")
    sid = lax.axis_index("s")
    wid = sid * 2 + cid
    base = SC_BASE + wid * SC_R

    lane16 = lax.iota(jnp.int32, 16)

    def start(gi, b):
        rvec = lane16 + jax.lax.broadcast(base + gi * GSZ, (16,))
        pltpu.make_async_copy(nu_hbm.at[rvec], ubufs[b], sems[b]).start()
        pltpu.make_async_copy(nv_hbm.at[rvec], vbufs[b], sems[b]).start()

    def wait(b):
        pltpu.make_async_copy(nu_hbm.at[lane16], ubufs[b], sems[b]).wait()
        pltpu.make_async_copy(nv_hbm.at[lane16], vbufs[b], sems[b]).wait()

    def compute(b, acc_in):
        for t in range(GSZ // 4):
            svec = jnp.zeros((16,), jnp.float32)
            for q in range(4):
                r = t * 4 + q
                parts = []
                for j in range(EMB_DIM // 16):
                    uj = ubufs[b][r, pl.ds(16 * j, 16)]
                    vj = vbufs[b][r, pl.ds(16 * j, 16)]
                    parts.append(uj * vj)
                while len(parts) > 1:
                    parts = [parts[i] + parts[i + 1]
                             for i in range(0, len(parts), 2)]
                srow = jnp.sum(parts[0])
                svec = jnp.where(lane16 == q,
                                 jax.lax.broadcast(srow, (16,)), svec)
            s = jnp.clip(svec, -6.0, 6.0)
            y = s * s
            p = jnp.full((16,), _COEF[8], jnp.float32)
            for c in _COEF[7::-1]:
                p = p * y + jnp.full((16,), c, jnp.float32)
            out = 0.5 * s + p
            acc_in = acc_in + jnp.where(lane16 < 4, out,
                                        jnp.zeros((16,), jnp.float32))
        return acc_in

    for b in range(DEPTH):
        start(b, b)

    def ring(j, acc):
        for b in range(DEPTH):
            i = j * DEPTH + b
            wait(b)
            acc = compute(b, acc)

            @pl.when(i + DEPTH < SC_NGROUP)
            def _():
                start(i + DEPTH, b)
        return acc

    acc = lax.fori_loop(0, SC_NGROUP // DEPTH, ring,
                        jnp.zeros((16,), jnp.float32))
    accv[...] = acc
    pltpu.sync_copy(accv, out_hbm.at[wid])


def _sc_call(emb_neg_u, emb_neg_v):
    mesh = plsc.VectorSubcoreMesh(core_axis_name="c", subcore_axis_name="s",
                                  num_cores=2, num_subcores=16)
    scratch = ([pltpu.VMEM((GSZ, EMB_DIM), jnp.float32)] * (2 * DEPTH)
               + [pltpu.VMEM((16,), jnp.float32)]
               + [pltpu.SemaphoreType.DMA] * DEPTH)
    return pl.kernel(
        _sc_kernel_body,
        out_type=jax.ShapeDtypeStruct((NW, 16), jnp.float32),
        mesh=mesh,
        scratch_types=scratch,
        compiler_params=pltpu.CompilerParams(needs_layout_passes=False),
    )(emb_neg_u, emb_neg_v)


def kernel(emb_pos_u, emb_pos_v, emb_neg_u, emb_neg_v):
    sc_out = _sc_call(emb_neg_u, emb_neg_v)
    tc_tot = _tc_call(emb_pos_u, emb_pos_v, emb_neg_u, emb_neg_v)
    return (tc_tot[0, 0] + jnp.sum(sc_out)) / jnp.float32(N_POS)


# speedup vs baseline: 2.9400x; 1.7131x over previous
import functools

import jax
import jax.numpy as jnp
from jax import lax
from jax.experimental import pallas as pl
from jax.experimental.pallas import tpu as pltpu
from jax.experimental.pallas import tpu_sc as plsc

EMB_DIM = 128
N_POS = 128 * 370
NEGATIVE_SIZE = 5
N_NEG = N_POS * NEGATIVE_SIZE

SC_NEG = 46080
TC_NEG = N_NEG - SC_NEG
GRID = 10
POS_BLOCK = N_POS // GRID
NEG_BLOCK = TC_NEG // GRID

NW = 32
SC_R = SC_NEG // NW
GSZ = 16
SC_NGROUP = SC_R // GSZ
DEPTH = 10
SC_BASE = TC_NEG

_COEF = (6.93186578e-01, 1.24800880e-01, -5.03562042e-03, 2.85647393e-04,
         -1.41225172e-05, 5.10029689e-07, -1.20037919e-08, 1.61785882e-10,
         -9.40153145e-13)


def _tc_loss_kernel(pu_ref, pv_ref, nu_ref, nv_ref, out_ref):
    step = pl.program_id(0)

    def body(u, v, sign):
        n = u.shape[0]
        prod = (u * v).reshape(n // 128, 128, EMB_DIM)
        prod_t = jnp.swapaxes(prod, 1, 2)
        score = jnp.sum(prod_t, axis=1)
        score = jnp.clip(score, -6.0, 6.0)
        return jnp.sum(jnp.log1p(jnp.exp(sign * score)))

    partial = (body(pu_ref[...], pv_ref[...], -1.0)
               + body(nu_ref[...], nv_ref[...], 1.0))

    @pl.when(step == 0)
    def _init():
        out_ref[0, 0] = partial

    @pl.when(step != 0)
    def _acc():
        out_ref[0, 0] += partial


def _tc_call(emb_pos_u, emb_pos_v, emb_neg_u, emb_neg_v):
    pos_spec = pl.BlockSpec((POS_BLOCK, EMB_DIM), lambda i: (i, 0))
    neg_spec = pl.BlockSpec((NEG_BLOCK, EMB_DIM), lambda i: (i, 0))
    return pl.pallas_call(
        _tc_loss_kernel,
        grid=(GRID,),
        in_specs=[pos_spec, pos_spec, neg_spec, neg_spec],
        out_specs=pl.BlockSpec((1, 1), lambda i: (0, 0),
                               memory_space=pltpu.SMEM),
        out_shape=jax.ShapeDtypeStruct((1, 1), jnp.float32),
    )(emb_pos_u, emb_pos_v, emb_neg_u, emb_neg_v)


def _sc_kernel_body(nu_hbm, nv_hbm, out_hbm, *scratch):
    ubufs = scratch[:DEPTH]
    vbufs = scratch[DEPTH:2 * DEPTH]
    accv = scratch[2 * DEPTH]
    sems = scratch[2 * DEPTH + 1:]
    cid = lax.axis_index("c")
    sid = lax.axis_index("s")
    wid = sid * 2 + cid
    base = SC_BASE + wid * SC_R

    lane16 = lax.iota(jnp.int32, 16)

    def start(gi, b):
        rvec = lane16 + jax.lax.broadcast(base + gi * GSZ, (16,))
        pltpu.make_async_copy(nu_hbm.at[rvec], ubufs[b], sems[b]).start()
        pltpu.make_async_copy(nv_hbm.at[rvec], vbufs[b], sems[b]).start()

    def wait(b):
        pltpu.make_async_copy(nu_hbm.at[lane16], ubufs[b], sems[b]).wait()
        pltpu.make_async_copy(nv_hbm.at[lane16], vbufs[b], sems[b]).wait()

    def compute(b, acc_in):
        for t in range(GSZ // 4):
            svec = jnp.zeros((16,), jnp.float32)
            for q in range(4):
                r = t * 4 + q
                parts = []
                for j in range(EMB_DIM // 16):
                    uj = ubufs[b][r, pl.ds(16 * j, 16)]
                    vj = vbufs[b][r, pl.ds(16 * j, 16)]
                    parts.append(uj * vj)
                while len(parts) > 1:
                    parts = [parts[i] + parts[i + 1]
                             for i in range(0, len(parts), 2)]
                srow = jnp.sum(parts[0])
                svec = jnp.where(lane16 == q,
                                 jax.lax.broadcast(srow, (16,)), svec)
            s = jnp.clip(svec, -6.0, 6.0)
            y = s * s
            p = jnp.full((16,), _COEF[8], jnp.float32)
            for c in _COEF[7::-1]:
                p = p * y + jnp.full((16,), c, jnp.float32)
            out = 0.5 * s + p
            acc_in = acc_in + jnp.where(lane16 < 4, out,
                                        jnp.zeros((16,), jnp.float32))
        return acc_in

    for b in range(DEPTH):
        start(b, b)

    def ring(j, acc):
        for b in range(DEPTH):
            i = j * DEPTH + b
            wait(b)
            acc = compute(b, acc)

            @pl.when(i + DEPTH < SC_NGROUP)
            def _():
                start(i + DEPTH, b)
        return acc

    acc = lax.fori_loop(0, SC_NGROUP // DEPTH, ring,
                        jnp.zeros((16,), jnp.float32))
    accv[...] = acc
    pltpu.sync_copy(accv, out_hbm.at[wid])


def _sc_call(emb_neg_u, emb_neg_v):
    mesh = plsc.VectorSubcoreMesh(core_axis_name="c", subcore_axis_name="s",
                                  num_cores=2, num_subcores=16)
    scratch = ([pltpu.VMEM((GSZ, EMB_DIM), jnp.float32)] * (2 * DEPTH)
               + [pltpu.VMEM((16,), jnp.float32)]
               + [pltpu.SemaphoreType.DMA] * DEPTH)
    return pl.kernel(
        _sc_kernel_body,
        out_type=jax.ShapeDtypeStruct((NW, 16), jnp.float32),
        mesh=mesh,
        scratch_types=scratch,
        compiler_params=pltpu.CompilerParams(needs_layout_passes=False),
    )(emb_neg_u, emb_neg_v)


def kernel(emb_pos_u, emb_pos_v, emb_neg_u, emb_neg_v):
    sc_out = _sc_call(emb_neg_u, emb_neg_v)
    tc_tot = _tc_call(emb_pos_u, emb_pos_v, emb_neg_u, emb_neg_v)
    return (tc_tot[0, 0] + jnp.sum(sc_out)) / jnp.float32(N_POS)


# speedup vs baseline: 2.9585x; 1.0063x over previous
import functools

import jax
import jax.numpy as jnp
from jax import lax
from jax.experimental import pallas as pl
from jax.experimental.pallas import tpu as pltpu
from jax.experimental.pallas import tpu_sc as plsc

EMB_DIM = 128
N_POS = 128 * 370
NEGATIVE_SIZE = 5
N_NEG = N_POS * NEGATIVE_SIZE

SC_NEG = 20480
TC_NEG = N_NEG - SC_NEG
GRID = 10
POS_BLOCK = N_POS // GRID
NEG_BLOCK = TC_NEG // GRID

NW = 32
SC_R = SC_NEG // NW
GSZ = 16
SC_NGROUP = SC_R // GSZ
DEPTH = 10
SC_BASE = TC_NEG

_COEF = (6.93186578e-01, 1.24800880e-01, -5.03562042e-03, 2.85647393e-04,
         -1.41225172e-05, 5.10029689e-07, -1.20037919e-08, 1.61785882e-10,
         -9.40153145e-13)


def _tc_loss_kernel(pu_ref, pv_ref, nu_ref, nv_ref, out_ref):
    step = pl.program_id(0)

    def body(u, v, sign):
        n = u.shape[0]
        prod = (u * v).reshape(n // 128, 128, EMB_DIM)
        prod_t = jnp.swapaxes(prod, 1, 2)
        score = jnp.sum(prod_t, axis=1)
        score = jnp.clip(score, -6.0, 6.0)
        return jnp.sum(jnp.log1p(jnp.exp(sign * score)))

    partial = (body(pu_ref[...], pv_ref[...], -1.0)
               + body(nu_ref[...], nv_ref[...], 1.0))

    @pl.when(step == 0)
    def _init():
        out_ref[0, 0] = partial

    @pl.when(step != 0)
    def _acc():
        out_ref[0, 0] += partial


def _tc_call(emb_pos_u, emb_pos_v, emb_neg_u, emb_neg_v):
    pos_spec = pl.BlockSpec((POS_BLOCK, EMB_DIM), lambda i: (i, 0))
    neg_spec = pl.BlockSpec((NEG_BLOCK, EMB_DIM), lambda i: (i, 0))
    return pl.pallas_call(
        _tc_loss_kernel,
        grid=(GRID,),
        in_specs=[pos_spec, pos_spec, neg_spec, neg_spec],
        out_specs=pl.BlockSpec((1, 1), lambda i: (0, 0),
                               memory_space=pltpu.SMEM),
        out_shape=jax.ShapeDtypeStruct((1, 1), jnp.float32),
    )(emb_pos_u, emb_pos_v, emb_neg_u, emb_neg_v)


def _sc_kernel_body(nu_hbm, nv_hbm, out_hbm, *scratch):
    ubufs = scratch[:DEPTH]
    vbufs = scratch[DEPTH:2 * DEPTH]
    accv = scratch[2 * DEPTH]
    sems = scratch[2 * DEPTH + 1:]
    cid = lax.axis_index("c")
    sid = lax.axis_index("s")
    wid = sid * 2 + cid
    base = SC_BASE + wid * SC_R

    lane16 = lax.iota(jnp.int32, 16)

    def start(gi, b):
        rvec = lane16 + jax.lax.broadcast(base + gi * GSZ, (16,))
        pltpu.make_async_copy(nu_hbm.at[rvec], ubufs[b], sems[b]).start()
        pltpu.make_async_copy(nv_hbm.at[rvec], vbufs[b], sems[b]).start()

    def wait(b):
        pltpu.make_async_copy(nu_hbm.at[lane16], ubufs[b], sems[b]).wait()
        pltpu.make_async_copy(nv_hbm.at[lane16], vbufs[b], sems[b]).wait()

    def compute(b, acc_in):
        for t in range(GSZ // 4):
            svec = jnp.zeros((16,), jnp.float32)
            for q in range(4):
                r = t * 4 + q
                parts = []
                for j in range(EMB_DIM // 16):
                    uj = ubufs[b][r, pl.ds(16 * j, 16)]
                    vj = vbufs[b][r, pl.ds(16 * j, 16)]
                    parts.append(uj * vj)
                while len(parts) > 1:
                    parts = [parts[i] + parts[i + 1]
                             for i in range(0, len(parts), 2)]
                srow = jnp.sum(parts[0])
                svec = jnp.where(lane16 == q,
                                 jax.lax.broadcast(srow, (16,)), svec)
            s = jnp.clip(svec, -6.0, 6.0)
            y = s * s
            p = jnp.full((16,), _COEF[8], jnp.float32)
            for c in _COEF[7::-1]:
                p = p * y + jnp.full((16,), c, jnp.float32)
            out = 0.5 * s + p
            acc_in = acc_in + jnp.where(lane16 < 4, out,
                                        jnp.zeros((16,), jnp.float32))
        return acc_in

    for b in range(DEPTH):
        start(b, b)

    def ring(j, acc):
        for b in range(DEPTH):
            i = j * DEPTH + b
            wait(b)
            acc = compute(b, acc)

            @pl.when(i + DEPTH < SC_NGROUP)
            def _():
                start(i + DEPTH, b)
        return acc

    acc = lax.fori_loop(0, SC_NGROUP // DEPTH, ring,
                        jnp.zeros((16,), jnp.float32))
    accv[...] = acc
    pltpu.sync_copy(accv, out_hbm.at[wid])


def _sc_call(emb_neg_u, emb_neg_v):
    mesh = plsc.VectorSubcoreMesh(core_axis_name="c", subcore_axis_name="s",
                                  num_cores=2, num_subcores=16)
    scratch = ([pltpu.VMEM((GSZ, EMB_DIM), jnp.float32)] * (2 * DEPTH)
               + [pltpu.VMEM((16,), jnp.float32)]
               + [pltpu.SemaphoreType.DMA] * DEPTH)
    return pl.kernel(
        _sc_kernel_body,
        out_type=jax.ShapeDtypeStruct((NW, 16), jnp.float32),
        mesh=mesh,
        scratch_types=scratch,
        compiler_params=pltpu.CompilerParams(needs_layout_passes=False),
    )(emb_neg_u, emb_neg_v)


def kernel(emb_pos_u, emb_pos_v, emb_neg_u, emb_neg_v):
    sc_out = _sc_call(emb_neg_u, emb_neg_v)
    tc_tot = _tc_call(emb_pos_u, emb_pos_v, emb_neg_u, emb_neg_v)
    return (tc_tot[0, 0] + jnp.sum(sc_out)) / jnp.float32(N_POS)


# speedup vs baseline: 3.5090x; 1.1861x over previous
import jax
import jax.numpy as jnp
from jax.experimental import pallas as pl
from jax.experimental.pallas import tpu as pltpu

EMB_DIM = 128
N_POS = 128 * 370
NEGATIVE_SIZE = 5
N_NEG = N_POS * NEGATIVE_SIZE
BLOCK = 1280
GRID = N_POS // BLOCK


def _loss_kernel(pu_ref, pv_ref, nu_ref, nv_ref, out_ref):
    step = pl.program_id(0)

    def body(u, v, sign):
        n = u.shape[0]
        prod = (u * v).reshape(n // 128, 128, EMB_DIM)
        prod_t = jnp.swapaxes(prod, 1, 2)
        score = jnp.sum(prod_t, axis=1)
        score = jnp.clip(score, -6.0, 6.0)
        return jnp.sum(jnp.log1p(jnp.exp(sign * score)))

    pos_part = body(pu_ref[...], pv_ref[...], -1.0)
    neg_part = body(nu_ref[...], nv_ref[...], 1.0)

    partial = pos_part + neg_part

    @pl.when(step == 0)
    def _init():
        out_ref[0, 0] = partial

    @pl.when(step != 0)
    def _acc():
        out_ref[0, 0] += partial


def kernel(emb_pos_u, emb_pos_v, emb_neg_u, emb_neg_v):
    pos_spec = pl.BlockSpec((BLOCK, EMB_DIM), lambda i: (i, 0))
    neg_spec = pl.BlockSpec((BLOCK * NEGATIVE_SIZE, EMB_DIM), lambda i: (i, 0))

    total = pl.pallas_call(
        _loss_kernel,
        grid=(GRID,),
        in_specs=[pos_spec, pos_spec, neg_spec, neg_spec],
        out_specs=pl.BlockSpec((1, 1), lambda i: (0, 0),
                               memory_space=pltpu.SMEM),
        out_shape=jax.ShapeDtypeStruct((1, 1), jnp.float32),
    )(emb_pos_u, emb_pos_v, emb_neg_u, emb_neg_v)

    return total[0, 0] / jnp.float32(N_POS)
